# baseline (speedup 1.0000x reference)
import functools

import jax
import jax.numpy as jnp
from jax import lax
from jax.experimental import pallas as pl
from jax.experimental.pallas import tpu as pltpu
from jax.experimental.pallas import tpu_sc as plsc

N = 10000
E = 320000
EMB = 128
NRBF = 20
RCUT = 5.0

NC = 2
NS = 16
NW = NC * NS
Q = 4
K = EMB // Q
TROW = 6 * K
RROW = 5 * K
AROW = 4 * K
NPAD = 10240
RPT = NPAD // NS
EPW = E // NW
B = 80
NBLK = EPW // B


def _node_pack_body(s_ref, v_ref, w1_ref, b1_ref, w2_ref, b2_ref, out_ref):
    s_blk = s_ref[...]
    h = lax.dot_general(s_blk, w1_ref[...], (((1,), (1,)), ((), ())),
                        preferred_element_type=jnp.float32) + b1_ref[...]
    h = h * (1.0 / (1.0 + jnp.exp(-h)))
    sp = lax.dot_general(h, w2_ref[...], (((1,), (1,)), ((), ())),
                         preferred_element_type=jnp.float32) + b2_ref[...]
    v_blk = v_ref[...]
    for q in range(Q):
        c = q * K
        out_ref[q] = jnp.concatenate(
            [sp[:, c:c + K], sp[:, EMB + c:EMB + c + K],
             sp[:, 2 * EMB + c:2 * EMB + c + K],
             v_blk[:, 0, c:c + K], v_blk[:, 1, c:c + K],
             v_blk[:, 2, c:c + K]], axis=1)


def _rbf_pack_body(r_ref, rh_ref, wr_ref, br_ref, out_ref, *, q):
    r = r_ref[...]
    ns = (lax.broadcasted_iota(jnp.int32, (1, NRBF), 1) + 1).astype(jnp.float32)
    rbf = jnp.sin(ns * (jnp.pi / RCUT) * r) / r
    lin = lax.dot_general(rbf, wr_ref[...], (((1,), (1,)), ((), ())),
                          preferred_element_type=jnp.float32) + br_ref[...]
    fc = 0.5 * (jnp.cos((jnp.pi / RCUT) * r) + 1.0)
    fc = fc * (r < RCUT).astype(jnp.float32)
    rp = lin * lin * fc
    rh = rh_ref[...]
    c = q * K
    r3 = rp[:, 2 * EMB + c:2 * EMB + c + K]
    out_ref[...] = jnp.concatenate(
        [rp[:, c:c + K], rp[:, EMB + c:EMB + c + K],
         rh[:, 0:1] * r3, rh[:, 1:2] * r3, rh[:, 2:3] * r3], axis=1)


def _sc_edge_body(t_hbm, r_hbm, src_hbm, dst_hbm, out_hbm,
                  sidx, didx, rows, rbf, msg, acc, sem):
    cid = lax.axis_index("c")
    sid = lax.axis_index("s")
    wid = sid * NC + cid

    zero = jnp.zeros((16,), jnp.float32)

    def _zero_row(i, carry):
        for l in range(AROW // 16):
            msg[i, pl.ds(l * 16, 16)] = zero
        return carry

    lax.fori_loop(0, B, _zero_row, 0)
    for blk in range(RPT // B):
        pltpu.sync_copy(msg, acc.at[pl.ds(sid * RPT + blk * B, B)])
    plsc.subcore_barrier()

    ebase = wid * EPW

    def _block(i, carry):
        base = ebase + i * B
        pltpu.sync_copy(src_hbm.at[pl.ds(base, B)], sidx)
        pltpu.sync_copy(dst_hbm.at[pl.ds(base, B)], didx)
        pltpu.async_copy(t_hbm.at[sidx], rows, sem).wait()
        pltpu.sync_copy(r_hbm.at[pl.ds(base, B)], rbf)

        def _edge(b, ecarry):
            for l in range(K // 16):
                o = l * 16
                r1 = rbf[b, pl.ds(o, 16)]
                r2 = rbf[b, pl.ds(K + o, 16)]
                sp1 = rows[b, pl.ds(o, 16)]
                sp2 = rows[b, pl.ds(K + o, 16)]
                sp3 = rows[b, pl.ds(2 * K + o, 16)]
                msg[b, pl.ds(o, 16)] = r2 * sp2
                dvv = r1 * sp1
                for d in range(3):
                    rd = rbf[b, pl.ds((2 + d) * K + o, 16)]
                    vd = rows[b, pl.ds((3 + d) * K + o, 16)]
                    msg[b, pl.ds((1 + d) * K + o, 16)] = vd * dvv + rd * sp3
            return ecarry

        lax.fori_loop(0, B, _edge, 0)
        pltpu.sync_copy(msg, acc.at[didx], add=True)
        return carry

    lax.fori_loop(0, NBLK, _block, 0)
    plsc.subcore_barrier()

    for blk in range(RPT // B):
        r0 = sid * RPT + blk * B
        pltpu.sync_copy(acc.at[pl.ds(r0, B)], msg)
        pltpu.sync_copy(msg, out_hbm.at[cid].at[pl.ds(r0, B)])


_sc_edge = functools.partial(
    pl.kernel,
    out_type=jax.ShapeDtypeStruct((NC, NPAD, AROW), jnp.float32),
    mesh=plsc.VectorSubcoreMesh(core_axis_name="c", subcore_axis_name="s",
                                num_cores=NC, num_subcores=NS),
    scratch_types=[
        pltpu.VMEM((B,), jnp.int32),
        pltpu.VMEM((B,), jnp.int32),
        pltpu.VMEM((B, TROW), jnp.float32),
        pltpu.VMEM((B, RROW), jnp.float32),
        pltpu.VMEM((B, AROW), jnp.float32),
        pltpu.VMEM_SHARED((NPAD, AROW), jnp.float32),
        pltpu.SemaphoreType.DMA,
    ],
    compiler_params=pltpu.CompilerParams(use_tc_tiling_on_sc=False),
)(_sc_edge_body)


BN = 1000
BE = 2000


def kernel(s, v, edges, r_ij, r_ij_normalized, W1, b1, W2, b2, Wr, br):
    t_tab = pl.pallas_call(
        _node_pack_body,
        grid=(N // BN,),
        in_specs=[
            pl.BlockSpec((BN, EMB), lambda i: (i, 0)),
            pl.BlockSpec((BN, 3, EMB), lambda i: (i, 0, 0)),
            pl.BlockSpec((EMB, EMB), lambda i: (0, 0)),
            pl.BlockSpec((1, EMB), lambda i: (0, 0)),
            pl.BlockSpec((3 * EMB, EMB), lambda i: (0, 0)),
            pl.BlockSpec((1, 3 * EMB), lambda i: (0, 0)),
        ],
        out_specs=pl.BlockSpec((Q, BN, TROW), lambda i: (0, i, 0)),
        out_shape=jax.ShapeDtypeStruct((Q, N, TROW), jnp.float32),
    )(s, v, W1, b1.reshape(1, EMB), W2, b2.reshape(1, 3 * EMB))

    t_flat = t_tab.reshape(Q * N, TROW)

    rh_pad = jnp.pad(r_ij_normalized, ((0, 0), (0, 13)))
    r_in = r_ij.reshape(E, 1)
    br_in = br.reshape(1, 3 * EMB)

    def _rbf_call(q):
        return pl.pallas_call(
            functools.partial(_rbf_pack_body, q=q),
            grid=(E // BE,),
            in_specs=[
                pl.BlockSpec((BE, 1), lambda i: (i, 0)),
                pl.BlockSpec((BE, 16), lambda i: (i, 0)),
                pl.BlockSpec((3 * EMB, NRBF), lambda i: (0, 0)),
                pl.BlockSpec((1, 3 * EMB), lambda i: (0, 0)),
            ],
            out_specs=pl.BlockSpec((BE, RROW), lambda i: (i, 0)),
            out_shape=jax.ShapeDtypeStruct((E, RROW), jnp.float32),
        )(r_in, rh_pad, Wr, br_in)

    dst = edges[:, 0]
    src = edges[:, 1]

    ds_parts, dv_parts = [], []
    for q in range(Q):
        r_q = _rbf_call(q)
        src_q = src + q * N
        part = _sc_edge(t_flat, r_q, src_q, dst)
        po = (part[0, :N] + part[1, :N])
        ds_parts.append(po[:, :K])
        dv_parts.append(po[:, K:])
    s_out = s + jnp.concatenate(ds_parts, axis=1)
    dv = jnp.stack(
        [jnp.concatenate([p[:, d * K:(d + 1) * K] for p in dv_parts], axis=1)
         for d in range(3)], axis=1)
    v_out = v + dv
    return (s_out, v_out)


# speedup vs baseline: 1.2898x; 1.2898x over previous
import functools

import jax
import jax.numpy as jnp
from jax import lax
from jax.experimental import pallas as pl
from jax.experimental.pallas import tpu as pltpu
from jax.experimental.pallas import tpu_sc as plsc

N = 10000
E = 320000
EMB = 128
NRBF = 20
RCUT = 5.0

NC = 2
NS = 16
NW = NC * NS
Q = 4
K = EMB // Q
TROW = 256
RROW = 128
AROW = 128
NPAD = 10240
RPT = NPAD // NS
EPW = E // NW
B = 40
NBLK = EPW // B


def _node_pack_body(s_ref, v_ref, w1_ref, b1_ref, w2p_ref, b2p_ref, out_ref):
    s_blk = s_ref[...]
    h = lax.dot_general(s_blk, w1_ref[...], (((1,), (1,)), ((), ())),
                        preferred_element_type=jnp.float32) + b1_ref[...]
    h = h * (1.0 / (1.0 + jnp.exp(-h)))
    sp = lax.dot_general(h, w2p_ref[...], (((1,), (1,)), ((), ())),
                         preferred_element_type=jnp.float32) + b2p_ref[...]
    v_blk = v_ref[...]
    zpad = jnp.zeros((s_blk.shape[0], K), jnp.float32)
    for q in range(Q):
        c = q * K
        out_ref[q] = jnp.concatenate(
            [sp[:, q * 128:(q + 1) * 128],
             v_blk[:, 0, c:c + K], v_blk[:, 1, c:c + K],
             v_blk[:, 2, c:c + K], zpad], axis=1)


def _rbf_pack_body(r_ref, rh_ref, wrp_ref, brp_ref, out_ref):
    r = r_ref[...]
    ns = (lax.broadcasted_iota(jnp.int32, (1, NRBF), 1) + 1).astype(jnp.float32)
    rbf = jnp.sin(ns * (jnp.pi / RCUT) * r) / r
    lin = lax.dot_general(rbf, wrp_ref[...], (((1,), (1,)), ((), ())),
                          preferred_element_type=jnp.float32) + brp_ref[...]
    fc = 0.5 * (jnp.cos((jnp.pi / RCUT) * r) + 1.0)
    fc = fc * (r < RCUT).astype(jnp.float32)
    rp = lin * lin * fc
    rh = rh_ref[...]
    be = r.shape[0]
    rh128 = jnp.concatenate(
        [jnp.zeros((be, 96), jnp.float32), rh,
         jnp.zeros((be, 16), jnp.float32)], axis=1)
    for q in range(Q):
        out_ref[q] = rp[:, q * 128:(q + 1) * 128] + rh128


def _sc_edge_body(t_hbm, r_hbm, src4_hbm, dst_hbm, out_hbm,
                  sidx, didx, rows, rbf, msg, zbuf, acc, sem):
    cid = lax.axis_index("c")
    sid = lax.axis_index("s")
    wid = sid * NC + cid

    zero = jnp.zeros((16,), jnp.float32)

    def _zero_row(i, carry):
        for l in range(AROW // 16):
            zbuf[i, pl.ds(l * 16, 16)] = zero
        return carry

    lax.fori_loop(0, B, _zero_row, 0)
    for blk in range(RPT // B):
        pltpu.sync_copy(zbuf, acc.at[pl.ds(sid * RPT + blk * B, B)])
    plsc.subcore_barrier()

    for q in range(Q):
        def _block(i, carry):
            ebase = wid * EPW + i * B
            pltpu.sync_copy(src4_hbm.at[pl.ds(q * E + ebase, B)], sidx)
            pltpu.sync_copy(dst_hbm.at[pl.ds(ebase, B)], didx)
            pltpu.async_copy(t_hbm.at[sidx], rows, sem).wait()
            pltpu.sync_copy(r_hbm.at[pl.ds(q * E + ebase, B)], rbf)

            def _edge(b, ecarry):
                bv = jnp.full((16,), b, jnp.int32)
                rh = [plsc.load_gather(
                          rbf, [bv, jnp.full((16,), 96 + d, jnp.int32)])
                      for d in range(3)]
                for l in range(K // 16):
                    o = l * 16
                    r1 = rbf[b, pl.ds(o, 16)]
                    r2 = rbf[b, pl.ds(K + o, 16)]
                    r3 = rbf[b, pl.ds(2 * K + o, 16)]
                    sp1 = rows[b, pl.ds(o, 16)]
                    sp2 = rows[b, pl.ds(K + o, 16)]
                    sp3 = rows[b, pl.ds(2 * K + o, 16)]
                    msg[b, pl.ds(o, 16)] = r2 * sp2
                    dvv = r1 * sp1
                    rep = r3 * sp3
                    for d in range(3):
                        vd = rows[b, pl.ds(128 + d * K + o, 16)]
                        msg[b, pl.ds((1 + d) * K + o, 16)] = (
                            vd * dvv + rh[d] * rep)
                return ecarry

            lax.fori_loop(0, B, _edge, 0)
            pltpu.sync_copy(msg, acc.at[didx], add=True)
            return carry

        lax.fori_loop(0, NBLK, _block, 0)
        plsc.subcore_barrier()

        obase = (q * NC + cid) * NPAD + sid * RPT
        for blk in range(RPT // B):
            pltpu.sync_copy(acc.at[pl.ds(sid * RPT + blk * B, B)], msg)
            pltpu.sync_copy(msg, out_hbm.at[pl.ds(obase + blk * B, B)])
            pltpu.sync_copy(zbuf, acc.at[pl.ds(sid * RPT + blk * B, B)])
        plsc.subcore_barrier()


_sc_edge = functools.partial(
    pl.kernel,
    out_type=jax.ShapeDtypeStruct((Q * NC * NPAD, AROW), jnp.float32),
    mesh=plsc.VectorSubcoreMesh(core_axis_name="c", subcore_axis_name="s",
                                num_cores=NC, num_subcores=NS),
    scratch_types=[
        pltpu.VMEM((B,), jnp.int32),
        pltpu.VMEM((B,), jnp.int32),
        pltpu.VMEM((B, TROW), jnp.float32),
        pltpu.VMEM((B, RROW), jnp.float32),
        pltpu.VMEM((B, AROW), jnp.float32),
        pltpu.VMEM((B, AROW), jnp.float32),
        pltpu.VMEM_SHARED((NPAD, AROW), jnp.float32),
        pltpu.SemaphoreType.DMA,
    ],
    compiler_params=pltpu.CompilerParams(needs_layout_passes=False),
)(_sc_edge_body)


BN = 1000
BE = 2000


def _permute_pad(w):
    parts = []
    zrow = jnp.zeros((K,) + w.shape[1:], w.dtype)
    for q in range(Q):
        c = q * K
        parts += [w[c:c + K], w[EMB + c:EMB + c + K],
                  w[2 * EMB + c:2 * EMB + c + K], zrow]
    return jnp.concatenate(parts, axis=0)


def kernel(s, v, edges, r_ij, r_ij_normalized, W1, b1, W2, b2, Wr, br):
    w2p = _permute_pad(W2)
    b2p = _permute_pad(b2.reshape(3 * EMB, 1)).reshape(1, 4 * EMB)
    wrp = _permute_pad(Wr)
    brp = _permute_pad(br.reshape(3 * EMB, 1)).reshape(1, 4 * EMB)

    t_tab = pl.pallas_call(
        _node_pack_body,
        grid=(N // BN,),
        in_specs=[
            pl.BlockSpec((BN, EMB), lambda i: (i, 0)),
            pl.BlockSpec((BN, 3, EMB), lambda i: (i, 0, 0)),
            pl.BlockSpec((EMB, EMB), lambda i: (0, 0)),
            pl.BlockSpec((1, EMB), lambda i: (0, 0)),
            pl.BlockSpec((4 * EMB, EMB), lambda i: (0, 0)),
            pl.BlockSpec((1, 4 * EMB), lambda i: (0, 0)),
        ],
        out_specs=pl.BlockSpec((Q, BN, TROW), lambda i: (0, i, 0)),
        out_shape=jax.ShapeDtypeStruct((Q, N, TROW), jnp.float32),
    )(s, v, W1, b1.reshape(1, EMB), w2p, b2p)

    rh_pad = jnp.pad(r_ij_normalized, ((0, 0), (0, 13)))
    r_tab = pl.pallas_call(
        _rbf_pack_body,
        grid=(E // BE,),
        in_specs=[
            pl.BlockSpec((BE, 1), lambda i: (i, 0)),
            pl.BlockSpec((BE, 16), lambda i: (i, 0)),
            pl.BlockSpec((4 * EMB, NRBF), lambda i: (0, 0)),
            pl.BlockSpec((1, 4 * EMB), lambda i: (0, 0)),
        ],
        out_specs=pl.BlockSpec((Q, BE, RROW), lambda i: (0, i, 0)),
        out_shape=jax.ShapeDtypeStruct((Q, E, RROW), jnp.float32),
    )(r_ij.reshape(E, 1), rh_pad, wrp, brp)

    dst = edges[:, 0]
    src = edges[:, 1]
    src4 = (src[None, :] + (jnp.arange(Q, dtype=jnp.int32) * N)[:, None]
            ).reshape(Q * E)

    out = _sc_edge(t_tab.reshape(Q * N, TROW), r_tab.reshape(Q * E, RROW),
                   src4, dst)
    out = out.reshape(Q, NC, NPAD, AROW)

    ds_parts, dv_parts = [], []
    for q in range(Q):
        po = out[q, 0, :N] + out[q, 1, :N]
        ds_parts.append(po[:, :K])
        dv_parts.append(po[:, K:])
    s_out = s + jnp.concatenate(ds_parts, axis=1)
    dv = jnp.stack(
        [jnp.concatenate([p[:, d * K:(d + 1) * K] for p in dv_parts], axis=1)
         for d in range(3)], axis=1)
    v_out = v + dv
    return (s_out, v_out)


# speedup vs baseline: 2.2721x; 1.7616x over previous
import functools

import jax
import jax.numpy as jnp
from jax import lax
from jax.experimental import pallas as pl
from jax.experimental.pallas import tpu as pltpu
from jax.experimental.pallas import tpu_sc as plsc

N = 10000
E = 320000
EMB = 128
NRBF = 20
RCUT = 5.0

NC = 2
NS = 16
NW = NC * NS
Q = 4
K = EMB // Q
TROW = 256
RROW = 128
AROW = 128
NPAD = 10240
RPT = NPAD // NS
EPW = E // NW
B = 40
NBLK = EPW // B


def _node_pack_body(s_ref, v_ref, w1_ref, b1_ref, w2p_ref, b2p_ref, out_ref):
    s_blk = s_ref[...]
    h = lax.dot_general(s_blk, w1_ref[...], (((1,), (1,)), ((), ())),
                        preferred_element_type=jnp.float32) + b1_ref[...]
    h = h * (1.0 / (1.0 + jnp.exp(-h)))
    sp = lax.dot_general(h, w2p_ref[...], (((1,), (1,)), ((), ())),
                         preferred_element_type=jnp.float32) + b2p_ref[...]
    v_blk = v_ref[...]
    zpad = jnp.zeros((s_blk.shape[0], K), jnp.float32)
    for q in range(Q):
        c = q * K
        out_ref[q] = jnp.concatenate(
            [sp[:, q * 128:(q + 1) * 128],
             v_blk[:, 0, c:c + K], v_blk[:, 1, c:c + K],
             v_blk[:, 2, c:c + K], zpad], axis=1)


def _rbf_pack_body(r_ref, rh_ref, wrp_ref, brp_ref, out_ref):
    r = r_ref[...]
    ns = (lax.broadcasted_iota(jnp.int32, (1, NRBF), 1) + 1).astype(jnp.float32)
    rbf = jnp.sin(ns * (jnp.pi / RCUT) * r) / r
    lin = lax.dot_general(rbf, wrp_ref[...], (((1,), (1,)), ((), ())),
                          preferred_element_type=jnp.float32) + brp_ref[...]
    fc = 0.5 * (jnp.cos((jnp.pi / RCUT) * r) + 1.0)
    fc = fc * (r < RCUT).astype(jnp.float32)
    rp = lin * lin * fc
    rh = rh_ref[...]
    be = r.shape[0]
    rh128 = jnp.concatenate(
        [jnp.zeros((be, 96), jnp.float32), rh,
         jnp.zeros((be, 16), jnp.float32)], axis=1)
    for q in range(Q):
        out_ref[q] = rp[:, q * 128:(q + 1) * 128] + rh128


def _sc_edge_body(t_hbm, r_hbm, src4_hbm, dst_hbm, out_hbm,
                  sidx, didx, rows, rbf, msg, zbuf, acc,
                  semi, semg, semr, sems):
    cid = lax.axis_index("c")
    sid = lax.axis_index("s")
    wid = sid * NC + cid

    zero = jnp.zeros((16,), jnp.float32)

    def _zero_row(i, carry):
        for l in range(AROW // 16):
            zbuf[i, pl.ds(l * 16, 16)] = zero
        return carry

    lax.fori_loop(0, B, _zero_row, 0)
    for blk in range(RPT // B):
        pltpu.sync_copy(zbuf, acc.at[pl.ds(sid * RPT + blk * B, B)])
    plsc.subcore_barrier()

    def _compute(p):
        def _edge(b, ecarry):
            bv = jnp.full((16,), b, jnp.int32)
            rh = [plsc.load_gather(
                      rbf, [jnp.full((16,), p, jnp.int32), bv,
                            jnp.full((16,), 96 + d, jnp.int32)])
                  for d in range(3)]
            for l in range(K // 16):
                o = l * 16
                r1 = rbf[p, b, pl.ds(o, 16)]
                r2 = rbf[p, b, pl.ds(K + o, 16)]
                r3 = rbf[p, b, pl.ds(2 * K + o, 16)]
                sp1 = rows[p, b, pl.ds(o, 16)]
                sp2 = rows[p, b, pl.ds(K + o, 16)]
                sp3 = rows[p, b, pl.ds(2 * K + o, 16)]
                msg[p, b, pl.ds(o, 16)] = r2 * sp2
                dvv = r1 * sp1
                rep = r3 * sp3
                for d in range(3):
                    vd = rows[p, b, pl.ds(128 + d * K + o, 16)]
                    msg[p, b, pl.ds((1 + d) * K + o, 16)] = (
                        vd * dvv + rh[d] * rep)
            return ecarry

        lax.fori_loop(0, B, _edge, 0)

    for q in range(Q):
        def _qbase(i):
            return q * E + wid * EPW + i * B, wid * EPW + i * B

        def _qstart_idx(i, j4):
            b4, be = _qbase(i)
            pltpu.async_copy(src4_hbm.at[pl.ds(b4, B)], sidx.at[j4],
                             semi.at[j4])
            pltpu.async_copy(dst_hbm.at[pl.ds(be, B)], didx.at[j4],
                             semi.at[j4])

        def _qwait_idx(i, j4):
            b4, be = _qbase(i)
            pltpu.make_async_copy(src4_hbm.at[pl.ds(b4, B)], sidx.at[j4],
                                  semi.at[j4]).wait()
            pltpu.make_async_copy(dst_hbm.at[pl.ds(be, B)], didx.at[j4],
                                  semi.at[j4]).wait()

        def _qstart_fetch(i, p, j4):
            b4, _ = _qbase(i)
            pltpu.async_copy(t_hbm.at[sidx.at[j4]], rows.at[p], semg.at[p])
            pltpu.async_copy(r_hbm.at[pl.ds(b4, B)], rbf.at[p], semr.at[p])

        def _qwait_fetch(i, p, j4):
            b4, _ = _qbase(i)
            pltpu.make_async_copy(t_hbm.at[sidx.at[j4]], rows.at[p],
                                  semg.at[p]).wait()
            pltpu.make_async_copy(r_hbm.at[pl.ds(b4, B)], rbf.at[p],
                                  semr.at[p]).wait()

        def _start_scat(p, j4):
            pltpu.async_copy(msg.at[p], acc.at[didx.at[j4]], sems.at[p],
                             add=True)

        def _wait_scat(p, j4):
            pltpu.make_async_copy(msg.at[p], acc.at[didx.at[j4]],
                                  sems.at[p]).wait()

        _qstart_idx(0, 0)
        _qstart_idx(1, 1)
        _qwait_idx(0, 0)
        _qstart_fetch(0, 0, 0)

        def _qpair(j, carry):
            t = 2 * j
            for p in range(2):
                tp = t + p

                @pl.when(tp + 1 < NBLK)
                def _():
                    _qwait_idx(tp + 1, (tp + 1) % 4)
                    _qstart_fetch(tp + 1, 1 - p, (tp + 1) % 4)

                @pl.when(tp >= 2)
                def _():
                    _wait_scat(p, (tp - 2) % 4)

                @pl.when(tp + 2 < NBLK)
                def _():
                    _qstart_idx(tp + 2, (tp + 2) % 4)

                _qwait_fetch(tp, p, tp % 4)
                _compute(p)
                _start_scat(p, tp % 4)

            return carry

        lax.fori_loop(0, NBLK // 2, _qpair, 0)
        _wait_scat(0, (NBLK - 2) % 4)
        _wait_scat(1, (NBLK - 1) % 4)
        plsc.subcore_barrier()

        obase = (q * NC + cid) * NPAD + sid * RPT
        for blk in range(RPT // B):
            pltpu.sync_copy(acc.at[pl.ds(sid * RPT + blk * B, B)],
                            msg.at[0])
            pltpu.sync_copy(msg.at[0], out_hbm.at[pl.ds(obase + blk * B, B)])
            pltpu.sync_copy(zbuf, acc.at[pl.ds(sid * RPT + blk * B, B)])
        plsc.subcore_barrier()


_sc_edge = functools.partial(
    pl.kernel,
    out_type=jax.ShapeDtypeStruct((Q * NC * NPAD, AROW), jnp.float32),
    mesh=plsc.VectorSubcoreMesh(core_axis_name="c", subcore_axis_name="s",
                                num_cores=NC, num_subcores=NS),
    scratch_types=[
        pltpu.VMEM((4, B), jnp.int32),
        pltpu.VMEM((4, B), jnp.int32),
        pltpu.VMEM((2, B, TROW), jnp.float32),
        pltpu.VMEM((2, B, RROW), jnp.float32),
        pltpu.VMEM((2, B, AROW), jnp.float32),
        pltpu.VMEM((B, AROW), jnp.float32),
        pltpu.VMEM_SHARED((NPAD, AROW), jnp.float32),
        pltpu.SemaphoreType.DMA((4,)),
        pltpu.SemaphoreType.DMA((2,)),
        pltpu.SemaphoreType.DMA((2,)),
        pltpu.SemaphoreType.DMA((2,)),
    ],
    compiler_params=pltpu.CompilerParams(needs_layout_passes=False),
)(_sc_edge_body)


BN = 1000
BE = 2000


def _permute_pad(w):
    parts = []
    zrow = jnp.zeros((K,) + w.shape[1:], w.dtype)
    for q in range(Q):
        c = q * K
        parts += [w[c:c + K], w[EMB + c:EMB + c + K],
                  w[2 * EMB + c:2 * EMB + c + K], zrow]
    return jnp.concatenate(parts, axis=0)


def kernel(s, v, edges, r_ij, r_ij_normalized, W1, b1, W2, b2, Wr, br):
    w2p = _permute_pad(W2)
    b2p = _permute_pad(b2.reshape(3 * EMB, 1)).reshape(1, 4 * EMB)
    wrp = _permute_pad(Wr)
    brp = _permute_pad(br.reshape(3 * EMB, 1)).reshape(1, 4 * EMB)

    t_tab = pl.pallas_call(
        _node_pack_body,
        grid=(N // BN,),
        in_specs=[
            pl.BlockSpec((BN, EMB), lambda i: (i, 0)),
            pl.BlockSpec((BN, 3, EMB), lambda i: (i, 0, 0)),
            pl.BlockSpec((EMB, EMB), lambda i: (0, 0)),
            pl.BlockSpec((1, EMB), lambda i: (0, 0)),
            pl.BlockSpec((4 * EMB, EMB), lambda i: (0, 0)),
            pl.BlockSpec((1, 4 * EMB), lambda i: (0, 0)),
        ],
        out_specs=pl.BlockSpec((Q, BN, TROW), lambda i: (0, i, 0)),
        out_shape=jax.ShapeDtypeStruct((Q, N, TROW), jnp.float32),
    )(s, v, W1, b1.reshape(1, EMB), w2p, b2p)

    rh_pad = jnp.pad(r_ij_normalized, ((0, 0), (0, 13)))
    r_tab = pl.pallas_call(
        _rbf_pack_body,
        grid=(E // BE,),
        in_specs=[
            pl.BlockSpec((BE, 1), lambda i: (i, 0)),
            pl.BlockSpec((BE, 16), lambda i: (i, 0)),
            pl.BlockSpec((4 * EMB, NRBF), lambda i: (0, 0)),
            pl.BlockSpec((1, 4 * EMB), lambda i: (0, 0)),
        ],
        out_specs=pl.BlockSpec((Q, BE, RROW), lambda i: (0, i, 0)),
        out_shape=jax.ShapeDtypeStruct((Q, E, RROW), jnp.float32),
    )(r_ij.reshape(E, 1), rh_pad, wrp, brp)

    dst = edges[:, 0]
    src = edges[:, 1]
    src4 = (src[None, :] + (jnp.arange(Q, dtype=jnp.int32) * N)[:, None]
            ).reshape(Q * E)

    out = _sc_edge(t_tab.reshape(Q * N, TROW), r_tab.reshape(Q * E, RROW),
                   src4, dst)
    out = out.reshape(Q, NC, NPAD, AROW)

    ds_parts, dv_parts = [], []
    for q in range(Q):
        po = out[q, 0, :N] + out[q, 1, :N]
        ds_parts.append(po[:, :K])
        dv_parts.append(po[:, K:])
    s_out = s + jnp.concatenate(ds_parts, axis=1)
    dv = jnp.stack(
        [jnp.concatenate([p[:, d * K:(d + 1) * K] for p in dv_parts], axis=1)
         for d in range(3)], axis=1)
    v_out = v + dv
    return (s_out, v_out)


# speedup vs baseline: 3.0064x; 1.3232x over previous
import functools

import jax
import jax.numpy as jnp
from jax import lax
from jax.experimental import pallas as pl
from jax.experimental.pallas import tpu as pltpu
from jax.experimental.pallas import tpu_sc as plsc

N = 10000
E = 320000
EMB = 128
NRBF = 20
RCUT = 5.0

NC = 2
NS = 16
NW = NC * NS
Q = 4
K = EMB // Q
TROW = 256
RROW = 128
AROW = 128
NPAD = 10240
RPT = NPAD // NS
EPW = E // NW
B = 40
NBLK = EPW // B


def _node_pack_body(s_ref, v_ref, w1_ref, b1_ref, w2p_ref, b2p_ref, out_ref):
    s_blk = s_ref[...]
    h = lax.dot_general(s_blk, w1_ref[...], (((1,), (1,)), ((), ())),
                        preferred_element_type=jnp.float32) + b1_ref[...]
    h = h * (1.0 / (1.0 + jnp.exp(-h)))
    sp = lax.dot_general(h, w2p_ref[...], (((1,), (1,)), ((), ())),
                         preferred_element_type=jnp.float32) + b2p_ref[...]
    v_blk = v_ref[...]
    zpad = jnp.zeros((s_blk.shape[0], K), jnp.float32)
    for q in range(Q):
        c = q * K
        out_ref[q] = jnp.concatenate(
            [sp[:, q * 128:(q + 1) * 128],
             v_blk[:, 0, c:c + K], v_blk[:, 1, c:c + K],
             v_blk[:, 2, c:c + K], zpad], axis=1)


_SIN_ODD = (1.0, -1.666666666667e-01, 8.333333333335e-03, -1.984126984022e-04,
            2.755731911059e-06, -2.505210315010e-08, 1.605891016760e-10,
            -7.645137880697e-13)


def _sin_2pi_frac(t):
    y = t - jnp.floor(t) - 0.5
    w = (2.0 * jnp.pi) * y
    w2 = w * w
    acc = jnp.full_like(w, _SIN_ODD[-1])
    for c in _SIN_ODD[-2::-1]:
        acc = acc * w2 + c
    return -(acc * w)


def _rbf_pack_body(r_ref, rh_ref, wrp_ref, brp_ref, out_ref):
    r = r_ref[...]
    ns = (lax.broadcasted_iota(jnp.int32, (1, NRBF), 1) + 1).astype(jnp.float32)
    ph = r * (0.5 / RCUT)
    rbf = _sin_2pi_frac(ns * ph) / r
    lin = lax.dot_general(rbf, wrp_ref[...], (((1,), (1,)), ((), ())),
                          preferred_element_type=jnp.float32) + brp_ref[...]
    fc = 0.5 * (_sin_2pi_frac(ph + 0.25) + 1.0)
    fc = fc * (r < RCUT).astype(jnp.float32)
    rp = lin * lin * fc
    rh = rh_ref[...]
    be = r.shape[0]
    rh128 = jnp.concatenate(
        [jnp.zeros((be, 96), jnp.float32), rh,
         jnp.zeros((be, 29), jnp.float32)], axis=1)
    for q in range(Q):
        out_ref[q] = rp[:, q * 128:(q + 1) * 128] + rh128


def _sc_edge_body(t_hbm, r_hbm, src4_hbm, dst_hbm, out_hbm,
                  sidx, didx, rows, rbf, msg, zbuf, acc,
                  semi, semg, semr, sems):
    cid = lax.axis_index("c")
    sid = lax.axis_index("s")
    wid = sid * NC + cid

    zero = jnp.zeros((16,), jnp.float32)

    def _zero_row(i, carry):
        for l in range(AROW // 16):
            zbuf[i, pl.ds(l * 16, 16)] = zero
        return carry

    lax.fori_loop(0, B, _zero_row, 0)
    for blk in range(RPT // B):
        pltpu.sync_copy(zbuf, acc.at[pl.ds(sid * RPT + blk * B, B)])
    plsc.subcore_barrier()

    def _compute(p):
        def _edge(b, ecarry):
            bv = jnp.full((16,), b, jnp.int32)
            rh = [plsc.load_gather(
                      rbf, [jnp.full((16,), p, jnp.int32), bv,
                            jnp.full((16,), 96 + d, jnp.int32)])
                  for d in range(3)]
            for l in range(K // 16):
                o = l * 16
                r1 = rbf[p, b, pl.ds(o, 16)]
                r2 = rbf[p, b, pl.ds(K + o, 16)]
                r3 = rbf[p, b, pl.ds(2 * K + o, 16)]
                sp1 = rows[p, b, pl.ds(o, 16)]
                sp2 = rows[p, b, pl.ds(K + o, 16)]
                sp3 = rows[p, b, pl.ds(2 * K + o, 16)]
                msg[p, b, pl.ds(o, 16)] = r2 * sp2
                dvv = r1 * sp1
                rep = r3 * sp3
                for d in range(3):
                    vd = rows[p, b, pl.ds(128 + d * K + o, 16)]
                    msg[p, b, pl.ds((1 + d) * K + o, 16)] = (
                        vd * dvv + rh[d] * rep)
            return ecarry

        lax.fori_loop(0, B, _edge, 0)

    for q in range(Q):
        tq = t_hbm.at[q]
        rq = r_hbm.at[q]

        def _qbase(i):
            return q * E + wid * EPW + i * B, wid * EPW + i * B

        def _qstart_idx(i, j4):
            b4, be = _qbase(i)
            pltpu.async_copy(src4_hbm.at[pl.ds(b4, B)], sidx.at[j4],
                             semi.at[j4])
            pltpu.async_copy(dst_hbm.at[pl.ds(be, B)], didx.at[j4],
                             semi.at[j4])

        def _qwait_idx(i, j4):
            b4, be = _qbase(i)
            pltpu.make_async_copy(src4_hbm.at[pl.ds(b4, B)], sidx.at[j4],
                                  semi.at[j4]).wait()
            pltpu.make_async_copy(dst_hbm.at[pl.ds(be, B)], didx.at[j4],
                                  semi.at[j4]).wait()

        def _qstart_fetch(i, p, j4):
            _, be = _qbase(i)
            pltpu.async_copy(tq.at[sidx.at[j4]], rows.at[p], semg.at[p])
            pltpu.async_copy(rq.at[pl.ds(be, B)], rbf.at[p], semr.at[p])

        def _qwait_fetch(i, p, j4):
            _, be = _qbase(i)
            pltpu.make_async_copy(tq.at[sidx.at[j4]], rows.at[p],
                                  semg.at[p]).wait()
            pltpu.make_async_copy(rq.at[pl.ds(be, B)], rbf.at[p],
                                  semr.at[p]).wait()

        def _start_scat(p, j4):
            pltpu.async_copy(msg.at[p], acc.at[didx.at[j4]], sems.at[p],
                             add=True)

        def _wait_scat(p, j4):
            pltpu.make_async_copy(msg.at[p], acc.at[didx.at[j4]],
                                  sems.at[p]).wait()

        _qstart_idx(0, 0)
        _qstart_idx(1, 1)
        _qwait_idx(0, 0)
        _qstart_fetch(0, 0, 0)

        def _qpair(j, carry):
            t = 2 * j
            for p in range(2):
                tp = t + p

                @pl.when(tp + 1 < NBLK)
                def _():
                    _qwait_idx(tp + 1, (tp + 1) % 4)
                    _qstart_fetch(tp + 1, 1 - p, (tp + 1) % 4)

                @pl.when(tp >= 2)
                def _():
                    _wait_scat(p, (tp - 2) % 4)

                @pl.when(tp + 2 < NBLK)
                def _():
                    _qstart_idx(tp + 2, (tp + 2) % 4)

                _qwait_fetch(tp, p, tp % 4)
                _compute(p)
                _start_scat(p, tp % 4)

            return carry

        lax.fori_loop(0, NBLK // 2, _qpair, 0)
        _wait_scat(0, (NBLK - 2) % 4)
        _wait_scat(1, (NBLK - 1) % 4)
        plsc.subcore_barrier()

        obase = (q * NC + cid) * NPAD + sid * RPT
        for blk in range(RPT // B):
            pltpu.sync_copy(acc.at[pl.ds(sid * RPT + blk * B, B)],
                            msg.at[0])
            pltpu.sync_copy(msg.at[0], out_hbm.at[pl.ds(obase + blk * B, B)])
            pltpu.sync_copy(zbuf, acc.at[pl.ds(sid * RPT + blk * B, B)])
        plsc.subcore_barrier()


_sc_edge = functools.partial(
    pl.kernel,
    out_type=jax.ShapeDtypeStruct((Q * NC * NPAD, AROW), jnp.float32),
    mesh=plsc.VectorSubcoreMesh(core_axis_name="c", subcore_axis_name="s",
                                num_cores=NC, num_subcores=NS),
    scratch_types=[
        pltpu.VMEM((4, B), jnp.int32),
        pltpu.VMEM((4, B), jnp.int32),
        pltpu.VMEM((2, B, TROW), jnp.float32),
        pltpu.VMEM((2, B, RROW), jnp.float32),
        pltpu.VMEM((2, B, AROW), jnp.float32),
        pltpu.VMEM((B, AROW), jnp.float32),
        pltpu.VMEM_SHARED((NPAD, AROW), jnp.float32),
        pltpu.SemaphoreType.DMA((4,)),
        pltpu.SemaphoreType.DMA((2,)),
        pltpu.SemaphoreType.DMA((2,)),
        pltpu.SemaphoreType.DMA((2,)),
    ],
    compiler_params=pltpu.CompilerParams(needs_layout_passes=False),
)(_sc_edge_body)


BN = 1000
BE = 2000


def _permute_pad(w):
    parts = []
    zrow = jnp.zeros((K,) + w.shape[1:], w.dtype)
    for q in range(Q):
        c = q * K
        parts += [w[c:c + K], w[EMB + c:EMB + c + K],
                  w[2 * EMB + c:2 * EMB + c + K], zrow]
    return jnp.concatenate(parts, axis=0)


def kernel(s, v, edges, r_ij, r_ij_normalized, W1, b1, W2, b2, Wr, br):
    w2p = _permute_pad(W2)
    b2p = _permute_pad(b2.reshape(3 * EMB, 1)).reshape(1, 4 * EMB)
    wrp = _permute_pad(Wr)
    brp = _permute_pad(br.reshape(3 * EMB, 1)).reshape(1, 4 * EMB)

    t_tab = pl.pallas_call(
        _node_pack_body,
        grid=(N // BN,),
        in_specs=[
            pl.BlockSpec((BN, EMB), lambda i: (i, 0)),
            pl.BlockSpec((BN, 3, EMB), lambda i: (i, 0, 0)),
            pl.BlockSpec((EMB, EMB), lambda i: (0, 0)),
            pl.BlockSpec((1, EMB), lambda i: (0, 0)),
            pl.BlockSpec((4 * EMB, EMB), lambda i: (0, 0)),
            pl.BlockSpec((1, 4 * EMB), lambda i: (0, 0)),
        ],
        out_specs=pl.BlockSpec((Q, BN, TROW), lambda i: (0, i, 0)),
        out_shape=jax.ShapeDtypeStruct((Q, N, TROW), jnp.float32),
    )(s, v, W1, b1.reshape(1, EMB), w2p, b2p)

    r_tab = pl.pallas_call(
        _rbf_pack_body,
        grid=(E // BE,),
        in_specs=[
            pl.BlockSpec((BE, 1), lambda i: (i, 0)),
            pl.BlockSpec((BE, 3), lambda i: (i, 0)),
            pl.BlockSpec((4 * EMB, NRBF), lambda i: (0, 0)),
            pl.BlockSpec((1, 4 * EMB), lambda i: (0, 0)),
        ],
        out_specs=pl.BlockSpec((Q, BE, RROW), lambda i: (0, i, 0)),
        out_shape=jax.ShapeDtypeStruct((Q, E, RROW), jnp.float32),
    )(r_ij.reshape(E, 1), r_ij_normalized, wrp, brp)

    dst = edges[:, 0]
    src = edges[:, 1]
    src4 = (src[None, :] + (jnp.arange(Q, dtype=jnp.int32) * N)[:, None]
            ).reshape(Q * E)

    out = _sc_edge(t_tab, r_tab, src4, dst)
    out = out.reshape(Q, NC, NPAD, AROW)

    ds_parts, dv_parts = [], []
    for q in range(Q):
        po = out[q, 0, :N] + out[q, 1, :N]
        ds_parts.append(po[:, :K])
        dv_parts.append(po[:, K:])
    s_out = s + jnp.concatenate(ds_parts, axis=1)
    dv = jnp.stack(
        [jnp.concatenate([p[:, d * K:(d + 1) * K] for p in dv_parts], axis=1)
         for d in range(3)], axis=1)
    v_out = v + dv
    return (s_out, v_out)


# speedup vs baseline: 3.0278x; 1.0071x over previous
import functools

import jax
import jax.numpy as jnp
from jax import lax
from jax.experimental import pallas as pl
from jax.experimental.pallas import tpu as pltpu
from jax.experimental.pallas import tpu_sc as plsc

N = 10000
E = 320000
EMB = 128
NRBF = 20
RCUT = 5.0

NC = 2
NS = 16
NW = NC * NS
Q = 4
K = EMB // Q
TROW = 256
RROW = 128
AROW = 128
NPAD = 10240
RPT = NPAD // NS
EPW = E // NW
B = 40
NBLK = EPW // B


def _node_pack_body(s_ref, v_ref, w1_ref, b1_ref, w2p_ref, b2p_ref, *out_refs):
    s_blk = s_ref[...]
    h = lax.dot_general(s_blk, w1_ref[...], (((1,), (1,)), ((), ())),
                        preferred_element_type=jnp.float32) + b1_ref[...]
    h = h * (1.0 / (1.0 + jnp.exp(-h)))
    sp = lax.dot_general(h, w2p_ref[...], (((1,), (1,)), ((), ())),
                         preferred_element_type=jnp.float32) + b2p_ref[...]
    v_blk = v_ref[...]
    zpad = jnp.zeros((s_blk.shape[0], K), jnp.float32)
    for q in range(Q):
        c = q * K
        out_refs[q][...] = jnp.concatenate(
            [sp[:, q * 128:(q + 1) * 128],
             v_blk[:, 0, c:c + K], v_blk[:, 1, c:c + K],
             v_blk[:, 2, c:c + K], zpad], axis=1)


_SIN_ODD = (1.0, -1.666666666667e-01, 8.333333333335e-03, -1.984126984022e-04,
            2.755731911059e-06, -2.505210315010e-08, 1.605891016760e-10,
            -7.645137880697e-13)


def _sin_2pi_frac(t):
    y = t - jnp.floor(t) - 0.5
    w = (2.0 * jnp.pi) * y
    w2 = w * w
    acc = jnp.full_like(w, _SIN_ODD[-1])
    for c in _SIN_ODD[-2::-1]:
        acc = acc * w2 + c
    return -(acc * w)


def _rbf_pack_body(r_ref, rh_ref, wrp_ref, brp_ref, *out_refs):
    r = r_ref[...]
    ns = (lax.broadcasted_iota(jnp.int32, (1, NRBF), 1) + 1).astype(jnp.float32)
    ph = r * (0.5 / RCUT)
    rbf = _sin_2pi_frac(ns * ph) / r
    lin = lax.dot_general(rbf, wrp_ref[...], (((1,), (1,)), ((), ())),
                          preferred_element_type=jnp.float32) + brp_ref[...]
    fc = 0.5 * (_sin_2pi_frac(ph + 0.25) + 1.0)
    fc = fc * (r < RCUT).astype(jnp.float32)
    rp = lin * lin * fc
    rh = rh_ref[...]
    be = r.shape[0]
    rh128 = jnp.concatenate(
        [jnp.zeros((be, 96), jnp.float32), rh,
         jnp.zeros((be, 29), jnp.float32)], axis=1)
    for q in range(Q):
        out_refs[q][...] = rp[:, q * 128:(q + 1) * 128] + rh128


def _sc_edge_body(t0, t1, t2, t3, r0, r1, r2, r3, src_hbm, dst_hbm, out_hbm,
                  sidx, didx, rows, rbf, msg, zbuf, acc,
                  semi, semg, semr, sems):
    cid = lax.axis_index("c")
    sid = lax.axis_index("s")
    wid = sid * NC + cid

    zero = jnp.zeros((16,), jnp.float32)

    def _zero_row(i, carry):
        for l in range(AROW // 16):
            zbuf[i, pl.ds(l * 16, 16)] = zero
        return carry

    lax.fori_loop(0, B, _zero_row, 0)
    for blk in range(RPT // B):
        pltpu.sync_copy(zbuf, acc.at[pl.ds(sid * RPT + blk * B, B)])
    plsc.subcore_barrier()

    def _compute(p):
        def _edge(b, ecarry):
            bv = jnp.full((16,), b, jnp.int32)
            rh = [plsc.load_gather(
                      rbf, [jnp.full((16,), p, jnp.int32), bv,
                            jnp.full((16,), 96 + d, jnp.int32)])
                  for d in range(3)]
            for l in range(K // 16):
                o = l * 16
                r1 = rbf[p, b, pl.ds(o, 16)]
                r2 = rbf[p, b, pl.ds(K + o, 16)]
                r3 = rbf[p, b, pl.ds(2 * K + o, 16)]
                sp1 = rows[p, b, pl.ds(o, 16)]
                sp2 = rows[p, b, pl.ds(K + o, 16)]
                sp3 = rows[p, b, pl.ds(2 * K + o, 16)]
                msg[p, b, pl.ds(o, 16)] = r2 * sp2
                dvv = r1 * sp1
                rep = r3 * sp3
                for d in range(3):
                    vd = rows[p, b, pl.ds(128 + d * K + o, 16)]
                    msg[p, b, pl.ds((1 + d) * K + o, 16)] = (
                        vd * dvv + rh[d] * rep)
            return ecarry

        lax.fori_loop(0, B, _edge, 0)

    for q, (tq, rq) in enumerate(((t0, r0), (t1, r1), (t2, r2), (t3, r3))):
        def _qbase(i):
            return wid * EPW + i * B

        def _qstart_idx(i, j4):
            be = _qbase(i)
            pltpu.async_copy(src_hbm.at[pl.ds(be, B)], sidx.at[j4],
                             semi.at[j4])
            pltpu.async_copy(dst_hbm.at[pl.ds(be, B)], didx.at[j4],
                             semi.at[j4])

        def _qwait_idx(i, j4):
            be = _qbase(i)
            pltpu.make_async_copy(src_hbm.at[pl.ds(be, B)], sidx.at[j4],
                                  semi.at[j4]).wait()
            pltpu.make_async_copy(dst_hbm.at[pl.ds(be, B)], didx.at[j4],
                                  semi.at[j4]).wait()

        def _qstart_fetch(i, p, j4):
            be = _qbase(i)
            pltpu.async_copy(tq.at[sidx.at[j4]], rows.at[p], semg.at[p])
            pltpu.async_copy(rq.at[pl.ds(be, B)], rbf.at[p], semr.at[p])

        def _qwait_fetch(i, p, j4):
            be = _qbase(i)
            pltpu.make_async_copy(tq.at[sidx.at[j4]], rows.at[p],
                                  semg.at[p]).wait()
            pltpu.make_async_copy(rq.at[pl.ds(be, B)], rbf.at[p],
                                  semr.at[p]).wait()

        def _start_scat(p, j4):
            pltpu.async_copy(msg.at[p], acc.at[didx.at[j4]], sems.at[p],
                             add=True)

        def _wait_scat(p, j4):
            pltpu.make_async_copy(msg.at[p], acc.at[didx.at[j4]],
                                  sems.at[p]).wait()

        _qstart_idx(0, 0)
        _qstart_idx(1, 1)
        _qwait_idx(0, 0)
        _qstart_fetch(0, 0, 0)

        def _qpair(j, carry):
            t = 2 * j
            for p in range(2):
                tp = t + p

                @pl.when(tp + 1 < NBLK)
                def _():
                    _qwait_idx(tp + 1, (tp + 1) % 4)
                    _qstart_fetch(tp + 1, 1 - p, (tp + 1) % 4)

                @pl.when(tp >= 2)
                def _():
                    _wait_scat(p, (tp - 2) % 4)

                @pl.when(tp + 2 < NBLK)
                def _():
                    _qstart_idx(tp + 2, (tp + 2) % 4)

                _qwait_fetch(tp, p, tp % 4)
                _compute(p)
                _start_scat(p, tp % 4)

            return carry

        lax.fori_loop(0, NBLK // 2, _qpair, 0)
        _wait_scat(0, (NBLK - 2) % 4)
        _wait_scat(1, (NBLK - 1) % 4)
        plsc.subcore_barrier()

        obase = (q * NC + cid) * NPAD + sid * RPT
        for blk in range(RPT // B):
            pltpu.sync_copy(acc.at[pl.ds(sid * RPT + blk * B, B)],
                            msg.at[0])
            pltpu.sync_copy(msg.at[0], out_hbm.at[pl.ds(obase + blk * B, B)])
            pltpu.sync_copy(zbuf, acc.at[pl.ds(sid * RPT + blk * B, B)])
        plsc.subcore_barrier()


_sc_edge = functools.partial(
    pl.kernel,
    out_type=jax.ShapeDtypeStruct((Q * NC * NPAD, AROW), jnp.float32),
    mesh=plsc.VectorSubcoreMesh(core_axis_name="c", subcore_axis_name="s",
                                num_cores=NC, num_subcores=NS),
    scratch_types=[
        pltpu.VMEM((4, B), jnp.int32),
        pltpu.VMEM((4, B), jnp.int32),
        pltpu.VMEM((2, B, TROW), jnp.float32),
        pltpu.VMEM((2, B, RROW), jnp.float32),
        pltpu.VMEM((2, B, AROW), jnp.float32),
        pltpu.VMEM((B, AROW), jnp.float32),
        pltpu.VMEM_SHARED((NPAD, AROW), jnp.float32),
        pltpu.SemaphoreType.DMA((4,)),
        pltpu.SemaphoreType.DMA((2,)),
        pltpu.SemaphoreType.DMA((2,)),
        pltpu.SemaphoreType.DMA((2,)),
    ],
    compiler_params=pltpu.CompilerParams(needs_layout_passes=False),
)(_sc_edge_body)


BN = 1000
BE = 2000


def _permute_pad(w):
    parts = []
    zrow = jnp.zeros((K,) + w.shape[1:], w.dtype)
    for q in range(Q):
        c = q * K
        parts += [w[c:c + K], w[EMB + c:EMB + c + K],
                  w[2 * EMB + c:2 * EMB + c + K], zrow]
    return jnp.concatenate(parts, axis=0)


def kernel(s, v, edges, r_ij, r_ij_normalized, W1, b1, W2, b2, Wr, br):
    w2p = _permute_pad(W2)
    b2p = _permute_pad(b2.reshape(3 * EMB, 1)).reshape(1, 4 * EMB)
    wrp = _permute_pad(Wr)
    brp = _permute_pad(br.reshape(3 * EMB, 1)).reshape(1, 4 * EMB)

    t_tab = pl.pallas_call(
        _node_pack_body,
        grid=(N // BN,),
        in_specs=[
            pl.BlockSpec((BN, EMB), lambda i: (i, 0)),
            pl.BlockSpec((BN, 3, EMB), lambda i: (i, 0, 0)),
            pl.BlockSpec((EMB, EMB), lambda i: (0, 0)),
            pl.BlockSpec((1, EMB), lambda i: (0, 0)),
            pl.BlockSpec((4 * EMB, EMB), lambda i: (0, 0)),
            pl.BlockSpec((1, 4 * EMB), lambda i: (0, 0)),
        ],
        out_specs=[pl.BlockSpec((BN, TROW), lambda i: (i, 0))] * Q,
        out_shape=[jax.ShapeDtypeStruct((N, TROW), jnp.float32)] * Q,
    )(s, v, W1, b1.reshape(1, EMB), w2p, b2p)

    r_tab = pl.pallas_call(
        _rbf_pack_body,
        grid=(E // BE,),
        in_specs=[
            pl.BlockSpec((BE, 1), lambda i: (i, 0)),
            pl.BlockSpec((BE, 3), lambda i: (i, 0)),
            pl.BlockSpec((4 * EMB, NRBF), lambda i: (0, 0)),
            pl.BlockSpec((1, 4 * EMB), lambda i: (0, 0)),
        ],
        out_specs=[pl.BlockSpec((BE, RROW), lambda i: (i, 0))] * Q,
        out_shape=[jax.ShapeDtypeStruct((E, RROW), jnp.float32)] * Q,
    )(r_ij.reshape(E, 1), r_ij_normalized, wrp, brp)

    dst = edges[:, 0]
    src = edges[:, 1]

    out = _sc_edge(*t_tab, *r_tab, src, dst)
    out = out.reshape(Q, NC, NPAD, AROW)

    ds_parts, dv_parts = [], []
    for q in range(Q):
        po = out[q, 0, :N] + out[q, 1, :N]
        ds_parts.append(po[:, :K])
        dv_parts.append(po[:, K:])
    s_out = s + jnp.concatenate(ds_parts, axis=1)
    dv = jnp.stack(
        [jnp.concatenate([p[:, d * K:(d + 1) * K] for p in dv_parts], axis=1)
         for d in range(3)], axis=1)
    v_out = v + dv
    return (s_out, v_out)


# speedup vs baseline: 4.3438x; 1.4347x over previous
import functools

import jax
import jax.numpy as jnp
from jax import lax
from jax.experimental import pallas as pl
from jax.experimental.pallas import tpu as pltpu
from jax.experimental.pallas import tpu_sc as plsc

N = 10000
E = 320000
EMB = 128
NRBF = 20
RCUT = 5.0

NC = 2
NS = 16
NW = NC * NS
Q = 4
K = EMB // Q
TROW = 256
RROW = 128
AROW = 128
NPAD = 10240
RPT = NPAD // NS
EPW = E // NW
B = 40
NBLK = EPW // B


def _node_pack_body(s_ref, v_ref, w1_ref, b1_ref, w2p_ref, b2p_ref, *out_refs):
    s_blk = s_ref[...]
    h = lax.dot_general(s_blk, w1_ref[...], (((1,), (1,)), ((), ())),
                        preferred_element_type=jnp.float32) + b1_ref[...]
    h = h * (1.0 / (1.0 + jnp.exp(-h)))
    sp = lax.dot_general(h, w2p_ref[...], (((1,), (1,)), ((), ())),
                         preferred_element_type=jnp.float32) + b2p_ref[...]
    v_blk = v_ref[...]
    zpad = jnp.zeros((s_blk.shape[0], K), jnp.float32)
    for q in range(Q):
        c = q * K
        out_refs[q][...] = jnp.concatenate(
            [sp[:, q * 128:(q + 1) * 128],
             v_blk[:, 0, c:c + K], v_blk[:, 1, c:c + K],
             v_blk[:, 2, c:c + K], zpad], axis=1)


_SIN_ODD = (1.0, -1.666666666667e-01, 8.333333333335e-03, -1.984126984022e-04,
            2.755731911059e-06, -2.505210315010e-08, 1.605891016760e-10,
            -7.645137880697e-13)


def _sin_2pi_frac(t):
    y = t - jnp.floor(t) - 0.5
    w = (2.0 * jnp.pi) * y
    w2 = w * w
    acc = jnp.full_like(w, _SIN_ODD[-1])
    for c in _SIN_ODD[-2::-1]:
        acc = acc * w2 + c
    return -(acc * w)


def _rbf_pack_body(r_ref, rh_ref, wrp_ref, brp_ref, *out_refs):
    r = r_ref[...]
    ns = (lax.broadcasted_iota(jnp.int32, (1, NRBF), 1) + 1).astype(jnp.float32)
    ph = r * (0.5 / RCUT)
    rbf = _sin_2pi_frac(ns * ph) / r
    lin = lax.dot_general(rbf, wrp_ref[...], (((1,), (1,)), ((), ())),
                          preferred_element_type=jnp.float32) + brp_ref[...]
    fc = 0.5 * (_sin_2pi_frac(ph + 0.25) + 1.0)
    fc = fc * (r < RCUT).astype(jnp.float32)
    rp = lin * lin * fc
    rh = rh_ref[...]
    be = r.shape[0]
    rh128 = jnp.concatenate(
        [jnp.zeros((be, 96), jnp.float32), rh,
         jnp.zeros((be, 29), jnp.float32)], axis=1)
    for q in range(Q):
        out_refs[q][...] = rp[:, q * 128:(q + 1) * 128] + rh128


def _sc_edge_body(t0, t1, t2, t3, r0, r1, r2, r3, src_hbm, dst_hbm, out_hbm,
                  sidx, didx, rows, rbf, msg, zbuf, acc,
                  semi, semg, semr, sems):
    cid = lax.axis_index("c")
    sid = lax.axis_index("s")
    wid = sid * NC + cid

    zero = jnp.zeros((16,), jnp.float32)

    def _zero_row(i, carry):
        for l in range(AROW // 16):
            zbuf[i, pl.ds(l * 16, 16)] = zero
        return carry

    lax.fori_loop(0, B, _zero_row, 0)
    for blk in range(RPT // B):
        pltpu.sync_copy(zbuf, acc.at[pl.ds(sid * RPT + blk * B, B)])
    plsc.subcore_barrier()

    def _compute(p):
        @plsc.parallel_loop(0, B, 1, unroll=4)
        def _edge(b):
            bv = jnp.full((16,), b, jnp.int32)
            rh = [plsc.load_gather(
                      rbf, [jnp.full((16,), p, jnp.int32), bv,
                            jnp.full((16,), 96 + d, jnp.int32)])
                  for d in range(3)]
            for l in range(K // 16):
                o = l * 16
                r1 = rbf[p, b, pl.ds(o, 16)]
                r2 = rbf[p, b, pl.ds(K + o, 16)]
                r3 = rbf[p, b, pl.ds(2 * K + o, 16)]
                sp1 = rows[p, b, pl.ds(o, 16)]
                sp2 = rows[p, b, pl.ds(K + o, 16)]
                sp3 = rows[p, b, pl.ds(2 * K + o, 16)]
                msg[p, b, pl.ds(o, 16)] = r2 * sp2
                dvv = r1 * sp1
                rep = r3 * sp3
                for d in range(3):
                    vd = rows[p, b, pl.ds(128 + d * K + o, 16)]
                    msg[p, b, pl.ds((1 + d) * K + o, 16)] = (
                        vd * dvv + rh[d] * rep)

    for q, (tq, rq) in enumerate(((t0, r0), (t1, r1), (t2, r2), (t3, r3))):
        def _qbase(i):
            return wid * EPW + i * B

        def _qstart_idx(i, j4):
            be = _qbase(i)
            pltpu.async_copy(src_hbm.at[pl.ds(be, B)], sidx.at[j4],
                             semi.at[j4])
            pltpu.async_copy(dst_hbm.at[pl.ds(be, B)], didx.at[j4],
                             semi.at[j4])

        def _qwait_idx(i, j4):
            be = _qbase(i)
            pltpu.make_async_copy(src_hbm.at[pl.ds(be, B)], sidx.at[j4],
                                  semi.at[j4]).wait()
            pltpu.make_async_copy(dst_hbm.at[pl.ds(be, B)], didx.at[j4],
                                  semi.at[j4]).wait()

        def _qstart_fetch(i, p, j4):
            be = _qbase(i)
            pltpu.async_copy(tq.at[sidx.at[j4]], rows.at[p], semg.at[p])
            pltpu.async_copy(rq.at[pl.ds(be, B)], rbf.at[p], semr.at[p])

        def _qwait_fetch(i, p, j4):
            be = _qbase(i)
            pltpu.make_async_copy(tq.at[sidx.at[j4]], rows.at[p],
                                  semg.at[p]).wait()
            pltpu.make_async_copy(rq.at[pl.ds(be, B)], rbf.at[p],
                                  semr.at[p]).wait()

        def _start_scat(p, j4):
            pltpu.async_copy(msg.at[p], acc.at[didx.at[j4]], sems.at[p],
                             add=True)

        def _wait_scat(p, j4):
            pltpu.make_async_copy(msg.at[p], acc.at[didx.at[j4]],
                                  sems.at[p]).wait()

        _qstart_idx(0, 0)
        _qstart_idx(1, 1)
        _qwait_idx(0, 0)
        _qstart_fetch(0, 0, 0)

        def _qpair(j, carry):
            t = 2 * j
            for p in range(2):
                tp = t + p

                @pl.when(tp + 1 < NBLK)
                def _():
                    _qwait_idx(tp + 1, (tp + 1) % 4)
                    _qstart_fetch(tp + 1, 1 - p, (tp + 1) % 4)

                @pl.when(tp >= 2)
                def _():
                    _wait_scat(p, (tp - 2) % 4)

                @pl.when(tp + 2 < NBLK)
                def _():
                    _qstart_idx(tp + 2, (tp + 2) % 4)

                _qwait_fetch(tp, p, tp % 4)
                _compute(p)
                _start_scat(p, tp % 4)

            return carry

        lax.fori_loop(0, NBLK // 2, _qpair, 0)
        _wait_scat(0, (NBLK - 2) % 4)
        _wait_scat(1, (NBLK - 1) % 4)
        plsc.subcore_barrier()

        obase = (q * NC + cid) * NPAD + sid * RPT
        for blk in range(RPT // B):
            pltpu.sync_copy(acc.at[pl.ds(sid * RPT + blk * B, B)],
                            msg.at[0])
            pltpu.sync_copy(msg.at[0], out_hbm.at[pl.ds(obase + blk * B, B)])
            pltpu.sync_copy(zbuf, acc.at[pl.ds(sid * RPT + blk * B, B)])
        plsc.subcore_barrier()


_sc_edge = functools.partial(
    pl.kernel,
    out_type=jax.ShapeDtypeStruct((Q * NC * NPAD, AROW), jnp.float32),
    mesh=plsc.VectorSubcoreMesh(core_axis_name="c", subcore_axis_name="s",
                                num_cores=NC, num_subcores=NS),
    scratch_types=[
        pltpu.VMEM((4, B), jnp.int32),
        pltpu.VMEM((4, B), jnp.int32),
        pltpu.VMEM((2, B, TROW), jnp.float32),
        pltpu.VMEM((2, B, RROW), jnp.float32),
        pltpu.VMEM((2, B, AROW), jnp.float32),
        pltpu.VMEM((B, AROW), jnp.float32),
        pltpu.VMEM_SHARED((NPAD, AROW), jnp.float32),
        pltpu.SemaphoreType.DMA((4,)),
        pltpu.SemaphoreType.DMA((2,)),
        pltpu.SemaphoreType.DMA((2,)),
        pltpu.SemaphoreType.DMA((2,)),
    ],
    compiler_params=pltpu.CompilerParams(needs_layout_passes=False),
)(_sc_edge_body)


BN = 1000
BE = 2000


def _permute_pad(w):
    parts = []
    zrow = jnp.zeros((K,) + w.shape[1:], w.dtype)
    for q in range(Q):
        c = q * K
        parts += [w[c:c + K], w[EMB + c:EMB + c + K],
                  w[2 * EMB + c:2 * EMB + c + K], zrow]
    return jnp.concatenate(parts, axis=0)


def kernel(s, v, edges, r_ij, r_ij_normalized, W1, b1, W2, b2, Wr, br):
    w2p = _permute_pad(W2)
    b2p = _permute_pad(b2.reshape(3 * EMB, 1)).reshape(1, 4 * EMB)
    wrp = _permute_pad(Wr)
    brp = _permute_pad(br.reshape(3 * EMB, 1)).reshape(1, 4 * EMB)

    t_tab = pl.pallas_call(
        _node_pack_body,
        grid=(N // BN,),
        in_specs=[
            pl.BlockSpec((BN, EMB), lambda i: (i, 0)),
            pl.BlockSpec((BN, 3, EMB), lambda i: (i, 0, 0)),
            pl.BlockSpec((EMB, EMB), lambda i: (0, 0)),
            pl.BlockSpec((1, EMB), lambda i: (0, 0)),
            pl.BlockSpec((4 * EMB, EMB), lambda i: (0, 0)),
            pl.BlockSpec((1, 4 * EMB), lambda i: (0, 0)),
        ],
        out_specs=[pl.BlockSpec((BN, TROW), lambda i: (i, 0))] * Q,
        out_shape=[jax.ShapeDtypeStruct((N, TROW), jnp.float32)] * Q,
    )(s, v, W1, b1.reshape(1, EMB), w2p, b2p)

    r_tab = pl.pallas_call(
        _rbf_pack_body,
        grid=(E // BE,),
        in_specs=[
            pl.BlockSpec((BE, 1), lambda i: (i, 0)),
            pl.BlockSpec((BE, 3), lambda i: (i, 0)),
            pl.BlockSpec((4 * EMB, NRBF), lambda i: (0, 0)),
            pl.BlockSpec((1, 4 * EMB), lambda i: (0, 0)),
        ],
        out_specs=[pl.BlockSpec((BE, RROW), lambda i: (i, 0))] * Q,
        out_shape=[jax.ShapeDtypeStruct((E, RROW), jnp.float32)] * Q,
    )(r_ij.reshape(E, 1), r_ij_normalized, wrp, brp)

    dst = edges[:, 0]
    src = edges[:, 1]

    out = _sc_edge(*t_tab, *r_tab, src, dst)
    out = out.reshape(Q, NC, NPAD, AROW)

    ds_parts, dv_parts = [], []
    for q in range(Q):
        po = out[q, 0, :N] + out[q, 1, :N]
        ds_parts.append(po[:, :K])
        dv_parts.append(po[:, K:])
    s_out = s + jnp.concatenate(ds_parts, axis=1)
    dv = jnp.stack(
        [jnp.concatenate([p[:, d * K:(d + 1) * K] for p in dv_parts], axis=1)
         for d in range(3)], axis=1)
    v_out = v + dv
    return (s_out, v_out)


# speedup vs baseline: 4.9922x; 1.1493x over previous
import functools

import jax
import jax.numpy as jnp
from jax import lax
from jax.experimental import pallas as pl
from jax.experimental.pallas import tpu as pltpu
from jax.experimental.pallas import tpu_sc as plsc

N = 10000
E = 320000
EMB = 128
NRBF = 20
RCUT = 5.0

NC = 2
NS = 16
NW = NC * NS
Q = 4
K = EMB // Q
TROW = 256
RROW = 128
AROW = 128
NPAD = 10240
RPT = NPAD // NS
EPW = E // NW
B = 40
NBLK = EPW // B


def _node_pack_body(s_ref, v_ref, w1_ref, b1_ref, w2p_ref, b2p_ref, *out_refs):
    s_blk = s_ref[...]
    h = lax.dot_general(s_blk, w1_ref[...], (((1,), (1,)), ((), ())),
                        preferred_element_type=jnp.float32) + b1_ref[...]
    h = h * (1.0 / (1.0 + jnp.exp(-h)))
    sp = lax.dot_general(h, w2p_ref[...], (((1,), (1,)), ((), ())),
                         preferred_element_type=jnp.float32) + b2p_ref[...]
    v_blk = v_ref[...]
    zpad = jnp.zeros((s_blk.shape[0], K), jnp.float32)
    for q in range(Q):
        c = q * K
        out_refs[q][...] = jnp.concatenate(
            [sp[:, q * 128:(q + 1) * 128],
             v_blk[:, 0, c:c + K], v_blk[:, 1, c:c + K],
             v_blk[:, 2, c:c + K], zpad], axis=1)


_SIN_ODD = (1.0, -1.666666666667e-01, 8.333333333335e-03, -1.984126984022e-04,
            2.755731911059e-06, -2.505210315010e-08, 1.605891016760e-10,
            -7.645137880697e-13)


def _sin_2pi_frac(t):
    y = t - jnp.floor(t) - 0.5
    w = (2.0 * jnp.pi) * y
    w2 = w * w
    acc = jnp.full_like(w, _SIN_ODD[-1])
    for c in _SIN_ODD[-2::-1]:
        acc = acc * w2 + c
    return -(acc * w)


def _rbf_pack_body(r_ref, rh_ref, wrp_ref, brp_ref, *out_refs):
    r = jnp.transpose(r_ref[...], (1, 0))
    ns = (lax.broadcasted_iota(jnp.int32, (1, NRBF), 1) + 1).astype(jnp.float32)
    ph = r * (0.5 / RCUT)
    rbf = _sin_2pi_frac(ns * ph) / r
    lin = lax.dot_general(rbf, wrp_ref[...], (((1,), (1,)), ((), ())),
                          preferred_element_type=jnp.float32) + brp_ref[...]
    fc = 0.5 * (_sin_2pi_frac(ph + 0.25) + 1.0)
    fc = fc * (r < RCUT).astype(jnp.float32)
    rp = lin * lin * fc
    rh = rh_ref[...]
    be = r.shape[0]
    rh128 = jnp.concatenate(
        [jnp.zeros((be, 96), jnp.float32), rh,
         jnp.zeros((be, 29), jnp.float32)], axis=1)
    for q in range(Q):
        out_refs[q][...] = rp[:, q * 128:(q + 1) * 128] + rh128


def _sc_edge_body(t0, t1, t2, t3, r0, r1, r2, r3, src_hbm, dst_hbm, out_hbm,
                  sidx, didx, rows, rbf, msg, zbuf, acc,
                  semi, semg, semr, sems):
    cid = lax.axis_index("c")
    sid = lax.axis_index("s")
    wid = sid * NC + cid

    zero = jnp.zeros((16,), jnp.float32)

    def _zero_row(i, carry):
        for l in range(AROW // 16):
            zbuf[i, pl.ds(l * 16, 16)] = zero
        return carry

    lax.fori_loop(0, B, _zero_row, 0)
    for blk in range(RPT // B):
        pltpu.sync_copy(zbuf, acc.at[pl.ds(sid * RPT + blk * B, B)])
    plsc.subcore_barrier()

    def _compute(p):
        @plsc.parallel_loop(0, B, 1, unroll=8)
        def _edge(b):
            bv = jnp.full((16,), b, jnp.int32)
            rh = [plsc.load_gather(
                      rbf, [jnp.full((16,), p, jnp.int32), bv,
                            jnp.full((16,), 96 + d, jnp.int32)])
                  for d in range(3)]
            for l in range(K // 16):
                o = l * 16
                r1 = rbf[p, b, pl.ds(o, 16)]
                r2 = rbf[p, b, pl.ds(K + o, 16)]
                r3 = rbf[p, b, pl.ds(2 * K + o, 16)]
                sp1 = rows[p, b, pl.ds(o, 16)]
                sp2 = rows[p, b, pl.ds(K + o, 16)]
                sp3 = rows[p, b, pl.ds(2 * K + o, 16)]
                msg[p, b, pl.ds(o, 16)] = r2 * sp2
                dvv = r1 * sp1
                rep = r3 * sp3
                for d in range(3):
                    vd = rows[p, b, pl.ds(128 + d * K + o, 16)]
                    msg[p, b, pl.ds((1 + d) * K + o, 16)] = (
                        vd * dvv + rh[d] * rep)

    for q, (tq, rq) in enumerate(((t0, r0), (t1, r1), (t2, r2), (t3, r3))):
        def _qbase(i):
            return wid * EPW + i * B

        def _qstart_idx(i, j4):
            be = _qbase(i)
            pltpu.async_copy(src_hbm.at[pl.ds(be, B)], sidx.at[j4],
                             semi.at[j4])
            pltpu.async_copy(dst_hbm.at[pl.ds(be, B)], didx.at[j4],
                             semi.at[j4])

        def _qwait_idx(i, j4):
            be = _qbase(i)
            pltpu.make_async_copy(src_hbm.at[pl.ds(be, B)], sidx.at[j4],
                                  semi.at[j4]).wait()
            pltpu.make_async_copy(dst_hbm.at[pl.ds(be, B)], didx.at[j4],
                                  semi.at[j4]).wait()

        def _qstart_fetch(i, p, j4):
            be = _qbase(i)
            pltpu.async_copy(tq.at[sidx.at[j4]], rows.at[p], semg.at[p])
            pltpu.async_copy(rq.at[pl.ds(be, B)], rbf.at[p], semr.at[p])

        def _qwait_fetch(i, p, j4):
            be = _qbase(i)
            pltpu.make_async_copy(tq.at[sidx.at[j4]], rows.at[p],
                                  semg.at[p]).wait()
            pltpu.make_async_copy(rq.at[pl.ds(be, B)], rbf.at[p],
                                  semr.at[p]).wait()

        def _start_scat(p, j4):
            pltpu.async_copy(msg.at[p], acc.at[didx.at[j4]], sems.at[p],
                             add=True)

        def _wait_scat(p, j4):
            pltpu.make_async_copy(msg.at[p], acc.at[didx.at[j4]],
                                  sems.at[p]).wait()

        _qstart_idx(0, 0)
        _qstart_idx(1, 1)
        _qwait_idx(0, 0)
        _qstart_fetch(0, 0, 0)

        def _qpair(j, carry):
            t = 2 * j
            for p in range(2):
                tp = t + p

                @pl.when(tp + 1 < NBLK)
                def _():
                    _qwait_idx(tp + 1, (tp + 1) % 4)
                    _qstart_fetch(tp + 1, 1 - p, (tp + 1) % 4)

                @pl.when(tp >= 2)
                def _():
                    _wait_scat(p, (tp - 2) % 4)

                @pl.when(tp + 2 < NBLK)
                def _():
                    _qstart_idx(tp + 2, (tp + 2) % 4)

                _qwait_fetch(tp, p, tp % 4)
                _compute(p)
                _start_scat(p, tp % 4)

            return carry

        lax.fori_loop(0, NBLK // 2, _qpair, 0)
        _wait_scat(0, (NBLK - 2) % 4)
        _wait_scat(1, (NBLK - 1) % 4)
        plsc.subcore_barrier()

        obase = (q * NC + cid) * NPAD + sid * RPT
        for blk in range(RPT // B):
            pltpu.sync_copy(acc.at[pl.ds(sid * RPT + blk * B, B)],
                            msg.at[0])
            pltpu.sync_copy(msg.at[0], out_hbm.at[pl.ds(obase + blk * B, B)])
            pltpu.sync_copy(zbuf, acc.at[pl.ds(sid * RPT + blk * B, B)])
        plsc.subcore_barrier()


_sc_edge = functools.partial(
    pl.kernel,
    out_type=jax.ShapeDtypeStruct((Q * NC * NPAD, AROW), jnp.float32),
    mesh=plsc.VectorSubcoreMesh(core_axis_name="c", subcore_axis_name="s",
                                num_cores=NC, num_subcores=NS),
    scratch_types=[
        pltpu.VMEM((4, B), jnp.int32),
        pltpu.VMEM((4, B), jnp.int32),
        pltpu.VMEM((2, B, TROW), jnp.float32),
        pltpu.VMEM((2, B, RROW), jnp.float32),
        pltpu.VMEM((2, B, AROW), jnp.float32),
        pltpu.VMEM((B, AROW), jnp.float32),
        pltpu.VMEM_SHARED((NPAD, AROW), jnp.float32),
        pltpu.SemaphoreType.DMA((4,)),
        pltpu.SemaphoreType.DMA((2,)),
        pltpu.SemaphoreType.DMA((2,)),
        pltpu.SemaphoreType.DMA((2,)),
    ],
    compiler_params=pltpu.CompilerParams(needs_layout_passes=False),
)(_sc_edge_body)


BN = 1000
BE = 2560


def _permute_pad(w):
    parts = []
    zrow = jnp.zeros((K,) + w.shape[1:], w.dtype)
    for q in range(Q):
        c = q * K
        parts += [w[c:c + K], w[EMB + c:EMB + c + K],
                  w[2 * EMB + c:2 * EMB + c + K], zrow]
    return jnp.concatenate(parts, axis=0)


def kernel(s, v, edges, r_ij, r_ij_normalized, W1, b1, W2, b2, Wr, br):
    w2p = _permute_pad(W2)
    b2p = _permute_pad(b2.reshape(3 * EMB, 1)).reshape(1, 4 * EMB)
    wrp = _permute_pad(Wr)
    brp = _permute_pad(br.reshape(3 * EMB, 1)).reshape(1, 4 * EMB)

    t_tab = pl.pallas_call(
        _node_pack_body,
        grid=(N // BN,),
        in_specs=[
            pl.BlockSpec((BN, EMB), lambda i: (i, 0)),
            pl.BlockSpec((BN, 3, EMB), lambda i: (i, 0, 0)),
            pl.BlockSpec((EMB, EMB), lambda i: (0, 0)),
            pl.BlockSpec((1, EMB), lambda i: (0, 0)),
            pl.BlockSpec((4 * EMB, EMB), lambda i: (0, 0)),
            pl.BlockSpec((1, 4 * EMB), lambda i: (0, 0)),
        ],
        out_specs=[pl.BlockSpec((BN, TROW), lambda i: (i, 0))] * Q,
        out_shape=[jax.ShapeDtypeStruct((N, TROW), jnp.float32)] * Q,
    )(s, v, W1, b1.reshape(1, EMB), w2p, b2p)

    r_tab = pl.pallas_call(
        _rbf_pack_body,
        grid=(E // BE,),
        in_specs=[
            pl.BlockSpec((1, BE), lambda i: (0, i)),
            pl.BlockSpec((BE, 3), lambda i: (i, 0)),
            pl.BlockSpec((4 * EMB, NRBF), lambda i: (0, 0)),
            pl.BlockSpec((1, 4 * EMB), lambda i: (0, 0)),
        ],
        out_specs=[pl.BlockSpec((BE, RROW), lambda i: (i, 0))] * Q,
        out_shape=[jax.ShapeDtypeStruct((E, RROW), jnp.float32)] * Q,
    )(r_ij.reshape(1, E), r_ij_normalized, wrp, brp)

    edges_t = edges.T
    dst = edges_t[0]
    src = edges_t[1]

    out = _sc_edge(*t_tab, *r_tab, src, dst)
    out = out.reshape(Q, NC, NPAD, AROW)

    ds_parts, dv_parts = [], []
    for q in range(Q):
        po = out[q, 0, :N] + out[q, 1, :N]
        ds_parts.append(po[:, :K])
        dv_parts.append(po[:, K:])
    s_out = s + jnp.concatenate(ds_parts, axis=1)
    dv = jnp.stack(
        [jnp.concatenate([p[:, d * K:(d + 1) * K] for p in dv_parts], axis=1)
         for d in range(3)], axis=1)
    v_out = v + dv
    return (s_out, v_out)
